# baseline (speedup 1.0000x reference)
import functools

import jax
import jax.numpy as jnp
from jax import lax
from jax.experimental import pallas as pl
from jax.experimental.pallas import tpu as pltpu
from jax.experimental.pallas import tpu_sc as plsc

B, T, C = 4, 8192, 1024
KTOP = 8
L = 16
NSG = 8
CHUNK = 800
HALF = 400
NCHUNK = 10
TAIL = T - NCHUNK * CHUNK
LEAF = 8
NCAND = 10
NEG_INF = float("-inf")
MINKEY = -0x80000000

_SORT8 = [(0, 1), (2, 3), (4, 5), (6, 7),
          (0, 2), (1, 3), (4, 6), (5, 7),
          (1, 2), (5, 6),
          (0, 4), (1, 5), (2, 6), (3, 7),
          (2, 4), (3, 5),
          (1, 2), (3, 4), (5, 6)]


def _orderkey(v):
  b = lax.bitcast_convert_type(v, jnp.int32)
  return b ^ (lax.shift_right_arithmetic(b, 31) & jnp.int32(0x7FFFFFFF))


def _insert_kv(ts, ix, v, iv):
  m = [v > t for t in ts]
  nts, nix = list(ts), list(ix)
  for j in range(KTOP - 1, 0, -1):
    nts[j] = jnp.where(m[j], jnp.where(m[j - 1], ts[j - 1], v), ts[j])
    nix[j] = jnp.where(m[j], jnp.where(m[j - 1], ix[j - 1], iv), ix[j])
  nts[0] = jnp.where(m[0], v, ts[0])
  nix[0] = jnp.where(m[0], iv, ix[0])
  return nts, nix


def _kmax_body(x_hbm, out_hbm, buf0, buf1, bsv, csv, csi, obuf, sem0, sem1):
  cid = lax.axis_index("c")
  sid = lax.axis_index("s")
  wid = sid * 2 + cid
  b = wid // NSG
  c0 = (wid % NSG) * 128
  lane = lax.iota(jnp.int32, L)

  def src_half(stage, half):
    return x_hbm.at[b, pl.ds(stage * CHUNK + half * HALF, HALF),
                    pl.ds(c0, 128)]

  def init_sg(sg, carry):
    for j in range(KTOP):
      csv[j, pl.ds(sg * L, L)] = jnp.full((L,), NEG_INF, jnp.float32)
      csi[j, pl.ds(sg * L, L)] = jnp.zeros((L,), jnp.int32)
    return carry
  lax.fori_loop(0, NSG, init_sg, 0)

  def leaf_scan(sg, bst, hbuf, lb0, nlf):
    csl = sg * L

    def leaf_body(ll, bst):
      base = ll * LEAF
      bm = hbuf[base, pl.ds(csl, L)]
      for r in range(1, LEAF):
        bm = jnp.maximum(bm, hbuf[base + r, pl.ds(csl, L)])
      key = (_orderkey(bm) & jnp.int32(~0x7F)) | (jnp.int32(127) - lb0 - ll)
      nb = list(bst)
      nb[0] = jnp.maximum(bst[0], key)
      for j in range(1, NCAND):
        nb[j] = jnp.maximum(bst[j], jnp.minimum(key, bst[j - 1]))
      return tuple(nb)

    return lax.fori_loop(0, nlf, leaf_body, bst)

  def first_half(sg, carry):
    bst = tuple(jnp.full((L,), MINKEY, jnp.int32) for _ in range(NCAND))
    bst = leaf_scan(sg, bst, buf0, 0, HALF // LEAF)
    for j in range(NCAND):
      bsv[j, pl.ds(sg * L, L)] = bst[j]
    return carry

  def second_half_and_select(sg, carry, coff, lb0, nlf):
    csl = sg * L
    coff_v = jnp.full((L,), coff, jnp.int32)
    bst = tuple(bsv[j, pl.ds(csl, L)] for j in range(NCAND))
    bst = leaf_scan(sg, bst, buf1 if nlf == HALF // LEAF else buf0,
                    lb0, nlf)
    brow = [(jnp.int32(127) - (k & jnp.int32(0x7F))) * LEAF for k in bst]
    ts = [csv[j, pl.ds(csl, L)] for j in range(KTOP)]
    ix = [csi[j, pl.ds(csl, L)] for j in range(KTOP)]

    def cand_body(r, st):
      ts, ix = list(st[:KTOP]), list(st[KTOP:])
      rv = jnp.full((L,), r, jnp.int32)
      for j in range(NCAND):
        lrow = brow[j] + rv
        lo = lrow < HALF
        rla = jnp.where(lo, lrow, 0)
        rlb = jnp.where(lo, 0, lrow - HALF)
        va = plsc.load_gather(buf0, [rla, lane + csl])
        vb = plsc.load_gather(buf1, [rlb, lane + csl])
        v = jnp.where(lo, va, vb)
        ts, ix = _insert_kv(ts, ix, v, lrow + coff_v)
      return tuple(ts) + tuple(ix)

    st = lax.fori_loop(0, LEAF, cand_body, tuple(ts) + tuple(ix))
    for j in range(KTOP):
      csv[j, pl.ds(csl, L)] = st[j]
      csi[j, pl.ds(csl, L)] = st[KTOP + j]
    return carry

  pltpu.async_copy(src_half(0, 0), buf0, sem0)
  pltpu.async_copy(src_half(0, 1), buf1, sem1)

  def main_body(i, carry):
    pltpu.make_async_copy(src_half(i, 0), buf0, sem0).wait()
    lax.fori_loop(0, NSG, first_half, 0)
    pltpu.make_async_copy(src_half(i, 1), buf1, sem1).wait()
    lax.fori_loop(
        0, NSG,
        lambda sg, c: second_half_and_select(
            sg, c, i * CHUNK, HALF // LEAF, HALF // LEAF), 0)
    nxt = jnp.minimum(i + 1, NCHUNK - 1)
    pltpu.async_copy(src_half(nxt, 0), buf0, sem0)
    pltpu.async_copy(src_half(nxt, 1), buf1, sem1)
    return carry

  lax.fori_loop(0, NCHUNK, main_body, 0)
  pltpu.make_async_copy(src_half(NCHUNK - 1, 0), buf0, sem0).wait()
  pltpu.make_async_copy(src_half(NCHUNK - 1, 1), buf1, sem1).wait()

  pltpu.sync_copy(x_hbm.at[b, pl.ds(NCHUNK * CHUNK, TAIL), pl.ds(c0, 128)],
                  buf0.at[pl.ds(0, TAIL)])

  def tail_sg(sg, carry):
    for j in range(NCAND):
      bsv[j, pl.ds(sg * L, L)] = jnp.full((L,), MINKEY, jnp.int32)
    return carry
  lax.fori_loop(0, NSG, tail_sg, 0)
  lax.fori_loop(
      0, NSG,
      lambda sg, c: second_half_and_select(
          sg, c, NCHUNK * CHUNK, 0, TAIL // LEAF), 0)

  def fin_sg(sg, carry):
    ts = [csv[j, pl.ds(sg * L, L)] for j in range(KTOP)]
    ix = [csi[j, pl.ds(sg * L, L)] for j in range(KTOP)]
    for (a, d) in _SORT8:
      swap = ix[a] > ix[d]
      ix[a], ix[d] = (jnp.where(swap, ix[d], ix[a]),
                      jnp.where(swap, ix[a], ix[d]))
      ts[a], ts[d] = (jnp.where(swap, ts[d], ts[a]),
                      jnp.where(swap, ts[a], ts[d]))
    for j in range(KTOP):
      obuf[j, pl.ds(sg * L, L)] = ts[j]
    return carry
  lax.fori_loop(0, NSG, fin_sg, 0)

  pltpu.sync_copy(obuf, out_hbm.at[b, :, pl.ds(c0, 128)])


@functools.partial(jax.jit, static_argnames=("k",))
def _kmax(x, k):
  del k
  f = pl.kernel(
      _kmax_body,
      out_type=jax.ShapeDtypeStruct((B, KTOP, C), jnp.float32),
      mesh=plsc.VectorSubcoreMesh(core_axis_name="c", subcore_axis_name="s"),
      scratch_types=[
          pltpu.VMEM((HALF, 128), jnp.float32),
          pltpu.VMEM((HALF, 128), jnp.float32),
          pltpu.VMEM((16, NSG * L), jnp.int32),
          pltpu.VMEM((KTOP, NSG * L), jnp.float32),
          pltpu.VMEM((KTOP, NSG * L), jnp.int32),
          pltpu.VMEM((KTOP, 128), jnp.float32),
          pltpu.SemaphoreType.DMA,
          pltpu.SemaphoreType.DMA,
      ],
      compiler_params=pltpu.CompilerParams(needs_layout_passes=False),
  )
  return f(x)


def kernel(x, k):
  return _kmax(x, 8)
